# baseline (speedup 1.0000x reference)
import functools

import jax
import jax.numpy as jnp
from jax import lax
from jax.experimental import pallas as pl
from jax.experimental.pallas import tpu as pltpu
from jax.experimental.pallas import tpu_sc as plsc

BATCH = 4096
SEQ = 200
EMBED = 64
LANES = 16

NC, NS = 2, 16
NW = NC * NS
CHUNK = 100
FLAT = BATCH * SEQ
NCHUNKS = FLAT // CHUNK
CHUNKS_PER_W = NCHUNKS // NW
NBUF = 4
NOUTER = CHUNKS_PER_W // NBUF


def _emb_kernel(idx_hbm, table_hbm, pos_hbm, out_hbm,
                idx_v, pos_v, b0, b1, b2, b3,
                g0, g1, g2, g3, w0, w1, w2, w3):
    bufs = [b0, b1, b2, b3]
    gsems = [g0, g1, g2, g3]
    wsems = [w0, w1, w2, w3]

    c = lax.axis_index("c")
    s = lax.axis_index("s")
    wid = s * NC + c
    cbase = wid * CHUNKS_PER_W

    pltpu.sync_copy(idx_hbm.at[pl.ds(cbase, CHUNKS_PER_W)], idx_v)
    pltpu.sync_copy(pos_hbm, pos_v)

    def gather_start(i_local, b):
        pltpu.async_copy(table_hbm.at[idx_v.at[i_local]], bufs[b], gsems[b])

    def gather_wait(b):
        pltpu.make_async_copy(table_hbm.at[idx_v.at[0]], bufs[b], gsems[b]).wait()

    def wb_start(i_local, b):
        pltpu.async_copy(bufs[b], out_hbm.at[cbase + i_local], wsems[b])

    def wb_wait(b):
        pltpu.make_async_copy(bufs[b], out_hbm.at[cbase], wsems[b]).wait()

    for b in range(NBUF):
        gather_start(b, b)

    def outer(g, carry):
        for b in range(NBUF):
            i = g * NBUF + b
            gather_wait(b)
            pbase = (b % 2) * CHUNK
            buf = bufs[b]

            def addrow(r, inner):
                pr = pbase + r
                for c4 in range(EMBED // LANES):
                    sl = pl.ds(c4 * LANES, LANES)
                    buf[r, sl] = buf[r, sl] + pos_v[pr, sl]
                return inner

            lax.fori_loop(0, CHUNK, addrow, 0, unroll=2)
            wb_start(i, b)
        for b in range(NBUF):
            @pl.when(g + 1 < NOUTER)
            def _():
                wb_wait(b)
                gather_start((g + 1) * NBUF + b, b)
        return carry

    lax.fori_loop(0, NOUTER, outer, 0)
    for b in range(NBUF):
        wb_wait(b)


@jax.jit
def _emb_lookup(idx, token_table, position_table):
    mesh = plsc.VectorSubcoreMesh(core_axis_name="c", subcore_axis_name="s")
    f = functools.partial(
        pl.kernel,
        out_type=jax.ShapeDtypeStruct((NCHUNKS, CHUNK, EMBED), jnp.float32),
        mesh=mesh,
        scratch_types=[
            pltpu.VMEM((CHUNKS_PER_W, CHUNK), jnp.int32),
            pltpu.VMEM((SEQ, EMBED), jnp.float32),
        ] + [pltpu.VMEM((CHUNK, EMBED), jnp.float32) for _ in range(NBUF)]
          + [pltpu.SemaphoreType.DMA for _ in range(2 * NBUF)],
        compiler_params=pltpu.CompilerParams(use_tc_tiling_on_sc=False),
    )(_emb_kernel)
    return f(idx, token_table, position_table)


def kernel(inputs, token_table, position_table):
    idx = inputs.reshape(-1).astype(jnp.int32).reshape(NCHUNKS, CHUNK)
    out = _emb_lookup(idx, token_table, position_table)
    return out.reshape(BATCH, SEQ, EMBED)


# speedup vs baseline: 1.6860x; 1.6860x over previous
import functools

import jax
import jax.numpy as jnp
from jax import lax
from jax.experimental import pallas as pl
from jax.experimental.pallas import tpu as pltpu
from jax.experimental.pallas import tpu_sc as plsc

BATCH = 4096
SEQ = 200
EMBED = 64
PADW = 128
LANES = 16

NC, NS = 2, 16
NW = NC * NS
CHUNK = 100
FLAT = BATCH * SEQ
NCHUNKS = FLAT // CHUNK
CHUNKS_PER_W = NCHUNKS // NW
HALF = 2
NBUF = 2 * HALF
NOUTER = CHUNKS_PER_W // HALF


def _emb_kernel(idx_hbm, table_hbm, pos_hbm, out_hbm,
                idx_v, pos_v, b0, b1, b2, b3,
                g0, g1, g2, g3, w0, w1, w2, w3):
    bufs = [b0, b1, b2, b3]
    gsems = [g0, g1, g2, g3]
    wsems = [w0, w1, w2, w3]

    c = lax.axis_index("c")
    s = lax.axis_index("s")
    wid = s * NC + c
    cbase = wid * CHUNKS_PER_W

    pltpu.sync_copy(idx_hbm.at[pl.ds(cbase, CHUNKS_PER_W)], idx_v)
    pltpu.sync_copy(pos_hbm, pos_v)

    def gather_start(i_local, b):
        pltpu.async_copy(table_hbm.at[idx_v.at[i_local]], bufs[b], gsems[b])

    def gather_wait(b):
        pltpu.make_async_copy(table_hbm.at[idx_v.at[0]], bufs[b], gsems[b]).wait()

    def wb_start(i_local, b):
        pltpu.async_copy(
            bufs[b].at[:, pl.ds(0, EMBED)],
            out_hbm.at[pl.ds((cbase + i_local) * CHUNK, CHUNK), pl.ds(0, EMBED)],
            wsems[b])

    def wb_wait(b):
        pltpu.make_async_copy(
            bufs[b].at[:, pl.ds(0, EMBED)],
            out_hbm.at[pl.ds(0, CHUNK), pl.ds(0, EMBED)], wsems[b]).wait()

    def add_pos(i_local, b):
        pbase = (b % 2) * CHUNK
        buf = bufs[b]

        @plsc.parallel_loop(0, CHUNK, step=1, unroll=4)
        def _(r):
            pr = pbase + r
            for c4 in range(EMBED // LANES):
                sl = pl.ds(c4 * LANES, LANES)
                buf[r, sl] = buf[r, sl] + pos_v[pr, sl]

    for b in range(HALF):
        gather_start(b, b)

    def outer(t, carry):
        for phase in range(2):
            g = 2 * t + phase
            pb = phase * HALF
            ob = (1 - phase) * HALF
            for b in range(HALF):
                @pl.when(g + 1 < NOUTER)
                def _():
                    @pl.when(g >= 1)
                    def _():
                        wb_wait(ob + b)
                    gather_start((g + 1) * HALF + b, ob + b)
            for b in range(HALF):
                gather_wait(pb + b)
                add_pos(g * HALF + b, pb + b)
                wb_start(g * HALF + b, pb + b)
        return carry

    lax.fori_loop(0, NOUTER // 2, outer, 0)
    for b in range(NBUF):
        wb_wait(b)


@jax.jit
def _emb_lookup(idx, table_pad, position_table):
    mesh = plsc.VectorSubcoreMesh(core_axis_name="c", subcore_axis_name="s")
    f = functools.partial(
        pl.kernel,
        out_type=jax.ShapeDtypeStruct((FLAT, PADW), jnp.float32),
        mesh=mesh,
        scratch_types=[
            pltpu.VMEM((CHUNKS_PER_W, CHUNK), jnp.int32),
            pltpu.VMEM((SEQ, EMBED), jnp.float32),
        ] + [pltpu.VMEM((CHUNK, PADW), jnp.float32) for _ in range(NBUF)]
          + [pltpu.SemaphoreType.DMA for _ in range(2 * NBUF)],
        compiler_params=pltpu.CompilerParams(use_tc_tiling_on_sc=False),
    )(_emb_kernel)
    return f(idx, table_pad, position_table)


def kernel(inputs, token_table, position_table):
    idx = inputs.reshape(-1).astype(jnp.int32).reshape(NCHUNKS, CHUNK)
    table_pad = jnp.pad(token_table, ((0, 0), (0, PADW - EMBED)))
    out = _emb_lookup(idx, table_pad, position_table)
    return out[:, :EMBED].reshape(BATCH, SEQ, EMBED)


# speedup vs baseline: 2.0630x; 1.2236x over previous
import functools

import jax
import jax.numpy as jnp
from jax import lax
from jax.experimental import pallas as pl
from jax.experimental.pallas import tpu as pltpu
from jax.experimental.pallas import tpu_sc as plsc

BATCH = 4096
SEQ = 200
EMBED = 64
PADW = 128
LANES = 16
VOCAB = 1000000
PBLK = 4096

NC, NS = 2, 16
NW = NC * NS
CHUNK = 100
FLAT = BATCH * SEQ
NCHUNKS = FLAT // CHUNK
CHUNKS_PER_W = NCHUNKS // NW
HALF = 2
NBUF = 2 * HALF
NOUTER = CHUNKS_PER_W // HALF


def _pad_body(t_ref, out_ref):
    x = t_ref[...]
    out_ref[:, :EMBED] = x.T
    out_ref[:, EMBED:] = jnp.zeros((PBLK, PADW - EMBED), jnp.float32)


def _pad_tc(t_transposed):
    return pl.pallas_call(
        _pad_body,
        grid=(pl.cdiv(VOCAB, PBLK),),
        in_specs=[pl.BlockSpec((EMBED, PBLK), lambda i: (0, i))],
        out_specs=pl.BlockSpec((PBLK, PADW), lambda i: (i, 0)),
        out_shape=jax.ShapeDtypeStruct((VOCAB, PADW), jnp.float32),
    )(t_transposed)


def _emb_kernel(idx_hbm, table_hbm, pos_hbm, out_hbm,
                idx_v, pos_v, b0, b1, b2, b3,
                g0, g1, g2, g3, w0, w1, w2, w3):
    bufs = [b0, b1, b2, b3]
    gsems = [g0, g1, g2, g3]
    wsems = [w0, w1, w2, w3]

    c = lax.axis_index("c")
    s = lax.axis_index("s")
    wid = s * NC + c
    cbase = wid * CHUNKS_PER_W

    pltpu.sync_copy(idx_hbm.at[pl.ds(cbase, CHUNKS_PER_W)], idx_v)
    pltpu.sync_copy(pos_hbm, pos_v)

    def gather_start(i_local, b):
        pltpu.async_copy(table_hbm.at[idx_v.at[i_local]], bufs[b], gsems[b])

    def gather_wait(b):
        pltpu.make_async_copy(table_hbm.at[idx_v.at[0]], bufs[b], gsems[b]).wait()

    def wb_start(i_local, b):
        pltpu.async_copy(
            bufs[b].at[:, pl.ds(0, EMBED)],
            out_hbm.at[pl.ds((cbase + i_local) * CHUNK, CHUNK), pl.ds(0, EMBED)],
            wsems[b])

    def wb_wait(b):
        pltpu.make_async_copy(
            bufs[b].at[:, pl.ds(0, EMBED)],
            out_hbm.at[pl.ds(0, CHUNK), pl.ds(0, EMBED)], wsems[b]).wait()

    def add_pos(i_local, b):
        pbase = (b % 2) * CHUNK
        buf = bufs[b]

        @plsc.parallel_loop(0, CHUNK, step=1, unroll=4)
        def _(r):
            pr = pbase + r
            for c4 in range(EMBED // LANES):
                sl = pl.ds(c4 * LANES, LANES)
                buf[r, sl] = buf[r, sl] + pos_v[pr, sl]

    for b in range(HALF):
        gather_start(b, b)

    def outer(t, carry):
        for phase in range(2):
            g = 2 * t + phase
            pb = phase * HALF
            ob = (1 - phase) * HALF
            for b in range(HALF):
                @pl.when(g + 1 < NOUTER)
                def _():
                    @pl.when(g >= 1)
                    def _():
                        wb_wait(ob + b)
                    gather_start((g + 1) * HALF + b, ob + b)
            for b in range(HALF):
                gather_wait(pb + b)
                add_pos(g * HALF + b, pb + b)
                wb_start(g * HALF + b, pb + b)
        return carry

    lax.fori_loop(0, NOUTER // 2, outer, 0)
    for b in range(NBUF):
        wb_wait(b)


@jax.jit
def _emb_lookup(idx, table_pad, position_table):
    mesh = plsc.VectorSubcoreMesh(core_axis_name="c", subcore_axis_name="s")
    f = functools.partial(
        pl.kernel,
        out_type=jax.ShapeDtypeStruct((FLAT, PADW), jnp.float32),
        mesh=mesh,
        scratch_types=[
            pltpu.VMEM((CHUNKS_PER_W, CHUNK), jnp.int32),
            pltpu.VMEM((SEQ, EMBED), jnp.float32),
        ] + [pltpu.VMEM((CHUNK, PADW), jnp.float32) for _ in range(NBUF)]
          + [pltpu.SemaphoreType.DMA for _ in range(2 * NBUF)],
        compiler_params=pltpu.CompilerParams(use_tc_tiling_on_sc=False),
    )(_emb_kernel)
    return f(idx, table_pad, position_table)


def kernel(inputs, token_table, position_table):
    idx = inputs.reshape(-1).astype(jnp.int32).reshape(NCHUNKS, CHUNK)
    table_pad = _pad_tc(token_table.T)
    out = _emb_lookup(idx, table_pad, position_table)
    return out[:, :EMBED].reshape(BATCH, SEQ, EMBED)


# speedup vs baseline: 3.3226x; 1.6105x over previous
import functools

import jax
import jax.numpy as jnp
from jax import lax
from jax.experimental import pallas as pl
from jax.experimental.pallas import tpu as pltpu
from jax.experimental.pallas import tpu_sc as plsc

BATCH = 4096
SEQ = 200
EMBED = 64
PADW = 128
LANES = 16
VOCAB = 1000000
PBLK = 8192

NC, NS = 2, 16
NW = NC * NS
CHUNK = 100
FLAT = BATCH * SEQ
NCHUNKS = FLAT // CHUNK
CHUNKS_PER_W = NCHUNKS // NW
HALF = 2
NBUF = 2 * HALF
NOUTER = CHUNKS_PER_W // HALF


def _pad_body(t_ref, out_ref):
    x = t_ref[...]
    out_ref[:, :EMBED] = x.T
    out_ref[:, EMBED:] = jnp.zeros((PBLK, PADW - EMBED), jnp.float32)


def _pad_tc(t_transposed):
    return pl.pallas_call(
        _pad_body,
        grid=(pl.cdiv(VOCAB, PBLK),),
        in_specs=[pl.BlockSpec((EMBED, PBLK), lambda i: (0, i))],
        out_specs=pl.BlockSpec((PBLK, PADW), lambda i: (i, 0)),
        out_shape=jax.ShapeDtypeStruct((VOCAB, PADW), jnp.float32),
    )(t_transposed)


def _emb_kernel(idx_hbm, table_hbm, pos_hbm, out_hbm,
                idx_v, pos_v, b0, b1, b2, b3,
                g0, g1, g2, g3, w0, w1, w2, w3):
    bufs = [b0, b1, b2, b3]
    gsems = [g0, g1, g2, g3]
    wsems = [w0, w1, w2, w3]

    c = lax.axis_index("c")
    s = lax.axis_index("s")
    wid = s * NC + c
    cbase = wid * CHUNKS_PER_W

    pltpu.sync_copy(idx_hbm.at[pl.ds(cbase, CHUNKS_PER_W)], idx_v)
    pltpu.sync_copy(pos_hbm, pos_v)

    def gather_start(i_local, b):
        pltpu.async_copy(table_hbm.at[idx_v.at[i_local]], bufs[b], gsems[b])

    def gather_wait(b):
        pltpu.make_async_copy(table_hbm.at[idx_v.at[0]], bufs[b], gsems[b]).wait()

    def wb_start(i_local, b):
        pltpu.async_copy(
            bufs[b].at[:, pl.ds(0, EMBED)],
            out_hbm.at[pl.ds((cbase + i_local) * CHUNK, CHUNK), pl.ds(0, EMBED)],
            wsems[b])

    def wb_wait(b):
        pltpu.make_async_copy(
            bufs[b].at[:, pl.ds(0, EMBED)],
            out_hbm.at[pl.ds(0, CHUNK), pl.ds(0, EMBED)], wsems[b]).wait()

    def add_pos(i_local, b):
        pbase = (b % 2) * CHUNK
        buf = bufs[b]

        @plsc.parallel_loop(0, CHUNK, step=1, unroll=4)
        def _(r):
            pr = pbase + r
            for c4 in range(EMBED // LANES):
                sl = pl.ds(c4 * LANES, LANES)
                buf[r, sl] = buf[r, sl] + pos_v[pr, sl]

    for b in range(HALF):
        gather_start(b, b)

    def outer(t, carry):
        for phase in range(2):
            g = 2 * t + phase
            pb = phase * HALF
            ob = (1 - phase) * HALF
            for b in range(HALF):
                @pl.when(g + 1 < NOUTER)
                def _():
                    @pl.when(g >= 1)
                    def _():
                        wb_wait(ob + b)
                    gather_start((g + 1) * HALF + b, ob + b)
            for b in range(HALF):
                gather_wait(pb + b)
                add_pos(g * HALF + b, pb + b)
                wb_start(g * HALF + b, pb + b)
        return carry

    lax.fori_loop(0, NOUTER // 2, outer, 0)
    for b in range(NBUF):
        wb_wait(b)


@jax.jit
def _emb_lookup(idx, table_pad, position_table):
    mesh = plsc.VectorSubcoreMesh(core_axis_name="c", subcore_axis_name="s")
    f = functools.partial(
        pl.kernel,
        out_type=jax.ShapeDtypeStruct((FLAT, PADW), jnp.float32),
        mesh=mesh,
        scratch_types=[
            pltpu.VMEM((CHUNKS_PER_W, CHUNK), jnp.int32),
            pltpu.VMEM((SEQ, EMBED), jnp.float32),
        ] + [pltpu.VMEM((CHUNK, PADW), jnp.float32) for _ in range(NBUF)]
          + [pltpu.SemaphoreType.DMA for _ in range(2 * NBUF)],
        compiler_params=pltpu.CompilerParams(use_tc_tiling_on_sc=False),
    )(_emb_kernel)
    return f(idx, table_pad, position_table)


def kernel(inputs, token_table, position_table):
    idx = inputs.reshape(-1).astype(jnp.int32).reshape(NCHUNKS, CHUNK)
    table_pad = _pad_tc(token_table.T)
    out = _emb_lookup(idx, table_pad, position_table)
    return out[:, :EMBED].reshape(BATCH, SEQ, EMBED)
